# baseline (speedup 1.0000x reference)
import functools
import math

import jax
import jax.numpy as jnp
from jax import lax
from jax.experimental import pallas as pl
from jax.experimental.pallas import tpu as pltpu
from jax.experimental.pallas import tpu_sc as plsc

D = 128
COEFF = math.sqrt(128.0)
LANES = 16

NC, NS = 2, 16
NW = NC * NS
B = 4096 * 50
BPW = B // NW
CHUNK = 320
GSUB = (128, 128, 64)
NB = 2
NCHUNK = BPW // CHUNK
TGROUP = NCHUNK // NB

_mesh = plsc.VectorSubcoreMesh(core_axis_name="c", subcore_axis_name="s")


@functools.partial(
    pl.kernel,
    mesh=_mesh,
    out_type=jax.ShapeDtypeStruct((B, D), jnp.float32),
    scratch_types=(
        [pltpu.VMEM((BPW,), jnp.int32)]
        + [pltpu.VMEM((CHUNK, D), jnp.float32) for _ in range(NB)]
        + [pltpu.SemaphoreType.DMA for _ in range(2 * NB)]
    ),
)
def _emb_lookup(table_hbm, idx_hbm, out_hbm, idx_v, *bufs_and_sems):
    bufs = bufs_and_sems[:NB]
    gsem = bufs_and_sems[NB:2 * NB]
    osem = bufs_and_sems[2 * NB:]

    wid = lax.axis_index("s") * NC + lax.axis_index("c")
    base = wid * BPW

    pltpu.sync_copy(idx_hbm.at[pl.ds(base, BPW)], idx_v)

    def gather(g, b):
        off = 0
        for n in GSUB:
            pltpu.make_async_copy(
                table_hbm.at[idx_v.at[pl.ds(g * CHUNK + off, n)]],
                bufs[b].at[pl.ds(off, n)],
                gsem[b],
            ).start()
            off += n

    def wait_gather(b):
        pltpu.make_async_copy(
            table_hbm.at[idx_v.at[pl.ds(0, CHUNK)]], bufs[b], gsem[b]
        ).wait()

    def put(g, b):
        pltpu.make_async_copy(
            bufs[b], out_hbm.at[pl.ds(base + g * CHUNK, CHUNK)], osem[b]
        ).start()

    def wait_put(b):
        pltpu.make_async_copy(
            bufs[b], out_hbm.at[pl.ds(base, CHUNK)], osem[b]
        ).wait()

    gather(0, 0)

    def body(t, carry):
        for k in range(NB):
            b = k
            bn = (k + 1) % NB
            g = t * NB + k
            @pl.when(g + 1 < NCHUNK)
            def _issue():
                @pl.when(g >= 1)
                def _drain():
                    wait_put(bn)
                gather(g + 1, bn)

            wait_gather(b)

            @plsc.parallel_loop(0, CHUNK, step=1, unroll=4)
            def row_body(i):
                for j in range(D // LANES):
                    sl = pl.ds(j * LANES, LANES)
                    bufs[b][i, sl] = bufs[b][i, sl] * COEFF

            put(g, b)
        return carry

    lax.fori_loop(0, TGROUP, body, 0, unroll=False)

    for b in range(NB):
        wait_put(b)


def kernel(x, table):
    idx = x.astype(jnp.int32).T.reshape(B)
    out = _emb_lookup(table, idx)
    return out.reshape(x.shape[1], x.shape[0], D).transpose(1, 0, 2)


# speedup vs baseline: 1.0304x; 1.0304x over previous
import functools
import math

import jax
import jax.numpy as jnp
from jax import lax
from jax.experimental import pallas as pl
from jax.experimental.pallas import tpu as pltpu
from jax.experimental.pallas import tpu_sc as plsc

D = 128
COEFF = math.sqrt(128.0)
LANES = 16

NC, NS = 2, 16
NW = NC * NS
B = 4096 * 50
BPW = B // NW
CHUNK = 128
NB = 5
NCHUNK = BPW // CHUNK
TGROUP = NCHUNK // NB

_mesh = plsc.VectorSubcoreMesh(core_axis_name="c", subcore_axis_name="s")


@functools.partial(
    pl.kernel,
    mesh=_mesh,
    out_type=jax.ShapeDtypeStruct((B, D), jnp.float32),
    scratch_types=(
        [pltpu.VMEM((BPW,), jnp.int32)]
        + [pltpu.VMEM((CHUNK, D), jnp.float32) for _ in range(NB)]
        + [pltpu.SemaphoreType.DMA for _ in range(2 * NB)]
    ),
)
def _emb_lookup(table_hbm, idx_hbm, out_hbm, idx_v, *bufs_and_sems):
    bufs = bufs_and_sems[:NB]
    gsem = bufs_and_sems[NB:2 * NB]
    osem = bufs_and_sems[2 * NB:]

    wid = lax.axis_index("s") * NC + lax.axis_index("c")
    base = wid * BPW

    pltpu.sync_copy(idx_hbm.at[pl.ds(base, BPW)], idx_v)

    def gather(g, b):
        pltpu.make_async_copy(
            table_hbm.at[idx_v.at[pl.ds(g * CHUNK, CHUNK)]],
            bufs[b],
            gsem[b],
        ).start()

    def wait_gather(b):
        pltpu.make_async_copy(
            table_hbm.at[idx_v.at[pl.ds(0, CHUNK)]], bufs[b], gsem[b]
        ).wait()

    def put(g, b):
        pltpu.make_async_copy(
            bufs[b], out_hbm.at[pl.ds(base + g * CHUNK, CHUNK)], osem[b]
        ).start()

    def wait_put(b):
        pltpu.make_async_copy(
            bufs[b], out_hbm.at[pl.ds(base, CHUNK)], osem[b]
        ).wait()

    LEAD = 4
    for b in range(LEAD):
        gather(b, b)

    def body(t, carry):
        for k in range(NB):
            b = k
            bn = (k + LEAD) % NB
            g = t * NB + k
            @pl.when(g + LEAD < NCHUNK)
            def _issue():
                @pl.when(g >= NB - LEAD)
                def _drain():
                    wait_put(bn)
                gather(g + LEAD, bn)

            wait_gather(b)

            def row_body(i, c):
                for j in range(D // LANES):
                    sl = pl.ds(j * LANES, LANES)
                    bufs[b][i, sl] = bufs[b][i, sl] * COEFF
                return c

            lax.fori_loop(0, CHUNK, row_body, 0, unroll=False)

            put(g, b)
        return carry

    lax.fori_loop(0, TGROUP, body, 0, unroll=False)

    for b in range(NB):
        wait_put(b)


def kernel(x, table):
    idx = x.astype(jnp.int32).T.reshape(B)
    out = _emb_lookup(table, idx)
    return out.reshape(x.shape[1], x.shape[0], D).transpose(1, 0, 2)
